# baseline (speedup 1.0000x reference)
import jax
import jax.numpy as jnp
from jax import lax
from jax.experimental import pallas as pl
from jax.experimental.pallas import tpu as pltpu
from jax.experimental.pallas import tpu_sc as plsc

D_MODEL = 64
SCALE = 8.0

NC = 2
NS = 16
NW = NC * NS

B = 4096
T_LEN = 200
TT = T_LEN // 8
BT = B // 128
DT = D_MODEL // 8
NBUF = 3


def _issue_gather(w_hbm, idxv, G, gsem, u, b):
    tt = lax.div(u, 8)
    t8 = lax.rem(u, 8)
    pltpu.async_copy(w_hbm.at[idxv.at[tt, t8]], G.at[b], gsem.at[b])


def _wait_gather(w_hbm, idxv, G, gsem, b):
    pltpu.make_async_copy(w_hbm.at[idxv.at[0, 0]], G.at[b], gsem.at[b]).wait()


def _body(xp_hbm, w_hbm, out_hbm, idxv, G, Tv, isem, gsem, wsem):
    c = lax.axis_index("c")
    s = lax.axis_index("s")
    bt = s * NC + c

    for tt in range(TT):
        pltpu.async_copy(xp_hbm.at[tt, bt], idxv.at[tt], isem)
    for tt in range(TT):
        pltpu.make_async_copy(xp_hbm.at[tt, bt], idxv.at[tt], isem).wait()

    iota16 = lax.iota(jnp.int32, 16)
    rids = [iota16 + (k * 16) for k in range(8)]

    _issue_gather(w_hbm, idxv, G, gsem, 0, 0)
    _issue_gather(w_hbm, idxv, G, gsem, 1, 1)

    def unit_body(u, carry):
        b = lax.rem(u, NBUF)
        _wait_gather(w_hbm, idxv, G, gsem, b)

        @pl.when(u >= NBUF)
        def _():
            for dt in range(DT):
                pltpu.make_async_copy(
                    Tv.at[b, pl.ds(dt * 1024, 1024)],
                    out_hbm.at[0, dt, bt],
                    wsem.at[b],
                ).wait()

        GB = G.at[b]
        TvB = Tv.at[b]
        for dc in range(D_MODEL):
            cid = jnp.full((16,), dc, jnp.int32)
            for k in range(8):
                v = plsc.load_gather(GB, [rids[k], cid])
                TvB[pl.ds(dc * 128 + k * 16, 16)] = v * SCALE

        @pl.when(u + 2 < T_LEN)
        def _():
            _issue_gather(w_hbm, idxv, G, gsem, u + 2, lax.rem(u + 2, NBUF))

        for dt in range(DT):
            pltpu.async_copy(
                Tv.at[b, pl.ds(dt * 1024, 1024)],
                out_hbm.at[u, dt, bt],
                wsem.at[b],
            )
        return carry

    lax.fori_loop(0, T_LEN, unit_body, 0)

    for b in range(NBUF):
        for dt in range(DT):
            pltpu.make_async_copy(
                Tv.at[b, pl.ds(dt * 1024, 1024)],
                out_hbm.at[0, dt, bt],
                wsem.at[b],
            ).wait()


@jax.jit
def kernel(x, W):
    if x.dtype != jnp.int32:
        x = x.astype(jnp.int32)
    xp = x.T.reshape(TT, 8, BT, 128).transpose(0, 2, 1, 3)
    mesh = plsc.VectorSubcoreMesh(core_axis_name="c", subcore_axis_name="s")
    out_phys = pl.kernel(
        _body,
        out_type=jax.ShapeDtypeStruct((T_LEN, DT, BT, 1024), jnp.float32),
        mesh=mesh,
        compiler_params=pltpu.CompilerParams(
            use_tc_tiling_on_sc=False, needs_layout_passes=False
        ),
        scratch_types=[
            pltpu.VMEM((TT, 8, 128), jnp.int32),
            pltpu.VMEM((NBUF, 128, D_MODEL), jnp.float32),
            pltpu.VMEM((NBUF, DT * 1024), jnp.float32),
            pltpu.SemaphoreType.DMA,
            pltpu.SemaphoreType.DMA((NBUF,)),
            pltpu.SemaphoreType.DMA((NBUF,)),
        ],
    )(xp, W)
    return (
        out_phys.reshape(T_LEN, DT, BT, 8, 128)
        .transpose(2, 4, 0, 1, 3)
        .reshape(B, T_LEN, D_MODEL)
    )


# speedup vs baseline: 1.4436x; 1.4436x over previous
import jax
import jax.numpy as jnp
from jax import lax
from jax.experimental import pallas as pl
from jax.experimental.pallas import tpu as pltpu
from jax.experimental.pallas import tpu_sc as plsc

D_MODEL = 64
SCALE = 8.0

NC = 2
NS = 16
NW = NC * NS

B = 4096
T_LEN = 200
TT = T_LEN // 8
BT = B // 128
DT = D_MODEL // 8
NBUF = 3


def _issue_gather(w_hbm, idxv, G, gsem, u, b):
    tt = lax.div(u, 8)
    t8 = lax.rem(u, 8)
    pltpu.async_copy(w_hbm.at[idxv.at[tt, t8]], G.at[b], gsem.at[b])


def _wait_gather(w_hbm, idxv, G, gsem, b):
    pltpu.make_async_copy(w_hbm.at[idxv.at[0, 0]], G.at[b], gsem.at[b]).wait()


def _body(xp_hbm, w_hbm, out_hbm, idxv, G, Tv, isem, gsem, wsem):
    c = lax.axis_index("c")
    s = lax.axis_index("s")
    bt = s * NC + c

    for tt in range(TT):
        pltpu.async_copy(xp_hbm.at[tt, bt], idxv.at[tt], isem)
    for tt in range(TT):
        pltpu.make_async_copy(xp_hbm.at[tt, bt], idxv.at[tt], isem).wait()

    iota16 = lax.iota(jnp.int32, 16)
    rids = [iota16 + (k * 16) for k in range(8)]

    _issue_gather(w_hbm, idxv, G, gsem, 0, 0)
    _issue_gather(w_hbm, idxv, G, gsem, 1, 1)

    def unit_body(u, carry):
        b = lax.rem(u, NBUF)
        _wait_gather(w_hbm, idxv, G, gsem, b)

        @pl.when(u >= NBUF)
        def _():
            for dt in range(DT):
                pltpu.make_async_copy(
                    Tv.at[b, pl.ds(dt * 1024, 1024)],
                    out_hbm.at[0, dt, bt],
                    wsem.at[b],
                ).wait()

        GB = G.at[b]
        TvB = Tv.at[b]

        @plsc.parallel_loop(0, 8)
        def _(ch):
            dc0 = ch * 8
            off0 = ch * 1024
            cid0 = jnp.full((16,), 0, jnp.int32) + dc0
            for dj in range(8):
                cid = cid0 + dj
                for k in range(8):
                    v = plsc.load_gather(GB, [rids[k], cid])
                    TvB[pl.ds(off0 + dj * 128 + k * 16, 16)] = v * SCALE

        @pl.when(u + 2 < T_LEN)
        def _():
            _issue_gather(w_hbm, idxv, G, gsem, u + 2, lax.rem(u + 2, NBUF))

        for dt in range(DT):
            pltpu.async_copy(
                Tv.at[b, pl.ds(dt * 1024, 1024)],
                out_hbm.at[u, dt, bt],
                wsem.at[b],
            )
        return carry

    lax.fori_loop(0, T_LEN, unit_body, 0)

    for b in range(NBUF):
        for dt in range(DT):
            pltpu.make_async_copy(
                Tv.at[b, pl.ds(dt * 1024, 1024)],
                out_hbm.at[0, dt, bt],
                wsem.at[b],
            ).wait()


@jax.jit
def kernel(x, W):
    if x.dtype != jnp.int32:
        x = x.astype(jnp.int32)
    xp = x.T.reshape(TT, 8, BT, 128).transpose(0, 2, 1, 3)
    mesh = plsc.VectorSubcoreMesh(core_axis_name="c", subcore_axis_name="s")
    out_phys = pl.kernel(
        _body,
        out_type=jax.ShapeDtypeStruct((T_LEN, DT, BT, 1024), jnp.float32),
        mesh=mesh,
        compiler_params=pltpu.CompilerParams(
            use_tc_tiling_on_sc=False, needs_layout_passes=False
        ),
        scratch_types=[
            pltpu.VMEM((TT, 8, 128), jnp.int32),
            pltpu.VMEM((NBUF, 128, D_MODEL), jnp.float32),
            pltpu.VMEM((NBUF, DT * 1024), jnp.float32),
            pltpu.SemaphoreType.DMA,
            pltpu.SemaphoreType.DMA((NBUF,)),
            pltpu.SemaphoreType.DMA((NBUF,)),
        ],
    )(xp, W)
    return (
        out_phys.reshape(T_LEN, DT, BT, 8, 128)
        .transpose(2, 4, 0, 1, 3)
        .reshape(B, T_LEN, D_MODEL)
    )


# speedup vs baseline: 2.4619x; 1.7054x over previous
import jax
import jax.numpy as jnp
from jax import lax
from jax.experimental import pallas as pl
from jax.experimental.pallas import tpu as pltpu
from jax.experimental.pallas import tpu_sc as plsc

D_MODEL = 64
SCALE = 8.0

NC = 2
NS = 16
NW = NC * NS

B = 4096
T_LEN = 200
TT = T_LEN // 8
BT = B // 128
DT = D_MODEL // 8
NBUF = 3


def _issue_gather(w_hbm, idxv, G, gsem, u, b):
    tt = lax.div(u, 8)
    t8 = lax.rem(u, 8)
    pltpu.async_copy(w_hbm.at[idxv.at[tt, t8]], G.at[b], gsem.at[b])


def _wait_gather(w_hbm, idxv, G, gsem, b):
    pltpu.make_async_copy(w_hbm.at[idxv.at[0, 0]], G.at[b], gsem.at[b]).wait()


def _body(xp_hbm, w_hbm, out_hbm, idxv, G, Gp, Tv, isem, gsem, wsem):
    c = lax.axis_index("c")
    s = lax.axis_index("s")
    bt = s * NC + c

    for tt in range(TT):
        pltpu.async_copy(xp_hbm.at[tt, bt], idxv.at[tt], isem)
    for tt in range(TT):
        pltpu.make_async_copy(xp_hbm.at[tt, bt], idxv.at[tt], isem).wait()

    iota16 = lax.iota(jnp.int32, 16)
    rids = [iota16 + (k * 16) for k in range(8)]
    zeros16 = jnp.full((16,), 0, jnp.int32)

    _issue_gather(w_hbm, idxv, G, gsem, 0, 0)
    _issue_gather(w_hbm, idxv, G, gsem, 1, 1)

    def unit_body(u, carry):
        b = lax.rem(u, NBUF)
        _wait_gather(w_hbm, idxv, G, gsem, b)

        @pl.when(u >= NBUF)
        def _():
            pltpu.make_async_copy(
                Tv.at[b], out_hbm.at[0, :, bt], wsem.at[b]
            ).wait()

        GB = G.at[b]
        TvB = Tv.at[b]

        @plsc.parallel_loop(0, 128, unroll=4)
        def _(bb):
            for k in range(4):
                sl = pl.ds(k * 16, 16)
                Gp[bb, sl] = GB[bb, sl] * SCALE

        @plsc.parallel_loop(0, D_MODEL, unroll=4)
        def _(dc):
            cid = zeros16 + dc
            dt = lax.shift_right_logical(dc, 3)
            off = lax.shift_left(lax.bitwise_and(dc, 7), 7)
            for k in range(8):
                v = plsc.load_gather(Gp, [rids[k], cid])
                TvB[dt, pl.ds(off + k * 16, 16)] = v

        @pl.when(u + 2 < T_LEN)
        def _():
            _issue_gather(w_hbm, idxv, G, gsem, u + 2, lax.rem(u + 2, NBUF))

        pltpu.async_copy(Tv.at[b], out_hbm.at[u, :, bt], wsem.at[b])
        return carry

    lax.fori_loop(0, T_LEN, unit_body, 0)

    for b in range(NBUF):
        pltpu.make_async_copy(
            Tv.at[b], out_hbm.at[0, :, bt], wsem.at[b]
        ).wait()


@jax.jit
def kernel(x, W):
    if x.dtype != jnp.int32:
        x = x.astype(jnp.int32)
    xp = x.T.reshape(TT, 8, BT, 128).transpose(0, 2, 1, 3)
    mesh = plsc.VectorSubcoreMesh(core_axis_name="c", subcore_axis_name="s")
    out_phys = pl.kernel(
        _body,
        out_type=jax.ShapeDtypeStruct((T_LEN, DT, BT, 1024), jnp.float32),
        mesh=mesh,
        compiler_params=pltpu.CompilerParams(
            use_tc_tiling_on_sc=False, needs_layout_passes=False
        ),
        scratch_types=[
            pltpu.VMEM((TT, 8, 128), jnp.int32),
            pltpu.VMEM((NBUF, 128, D_MODEL), jnp.float32),
            pltpu.VMEM((128, D_MODEL + 1), jnp.float32),
            pltpu.VMEM((NBUF, DT, 1024), jnp.float32),
            pltpu.SemaphoreType.DMA,
            pltpu.SemaphoreType.DMA((NBUF,)),
            pltpu.SemaphoreType.DMA((NBUF,)),
        ],
    )(xp, W)
    return (
        out_phys.reshape(T_LEN, DT, BT, 8, 128)
        .transpose(2, 4, 0, 1, 3)
        .reshape(B, T_LEN, D_MODEL)
    )
